# baseline (speedup 1.0000x reference)
import functools

import jax
import jax.numpy as jnp
from jax import lax
from jax.experimental import pallas as pl
from jax.experimental.pallas import tpu as pltpu
from jax.experimental.pallas import tpu_sc as plsc

VOCAB = 1000
HIDDEN = 64
B = 1024
L = 50

NC = 2
NS = 16
NW = NC * NS

B_TOT = B * L
BPW = B_TOT // NW
CHUNK = 64
NCHUNK = BPW // CHUNK


def _table_body(emb_ref, wt_ref, b_ref, out_ref):
    out_ref[...] = (
        jnp.dot(emb_ref[...], wt_ref[...], preferred_element_type=jnp.float32)
        + b_ref[...]
    )


def _build_table(emb_table, head_w_t, head_b_row):
    return pl.pallas_call(
        _table_body,
        out_shape=jax.ShapeDtypeStruct((VOCAB, VOCAB), jnp.float32),
    )(emb_table, head_w_t, head_b_row)


_mesh = plsc.VectorSubcoreMesh(
    core_axis_name="c", subcore_axis_name="s", num_cores=NC, num_subcores=NS
)


@functools.partial(
    pl.kernel,
    out_type=jax.ShapeDtypeStruct((B_TOT, VOCAB), jnp.float32),
    mesh=_mesh,
    scratch_types=[
        pltpu.VMEM((BPW,), jnp.int32),
        pltpu.VMEM((CHUNK, VOCAB), jnp.float32),
        pltpu.SemaphoreType.DMA,
    ],
    compiler_params=pltpu.CompilerParams(use_tc_tiling_on_sc=False),
)
def _gather_rows(table_hbm, idx_hbm, out_hbm, idx_v, rows_v, sem):
    wid = lax.axis_index("s") * NC + lax.axis_index("c")
    base = pl.multiple_of(wid * BPW, BPW)
    pltpu.sync_copy(idx_hbm.at[pl.ds(base, BPW)], idx_v)

    def body(c, carry):
        off = pl.multiple_of(c * CHUNK, CHUNK)
        pltpu.async_copy(
            table_hbm.at[idx_v.at[pl.ds(off, CHUNK)]], rows_v, sem
        ).wait()
        pltpu.sync_copy(rows_v, out_hbm.at[pl.ds(base + off, CHUNK)])
        return carry

    lax.fori_loop(0, NCHUNK, body, 0)


def kernel(input_ids, emb_table, head_w, head_b):
    table = _build_table(
        emb_table, head_w.T, head_b.reshape(1, VOCAB)
    )
    ids = input_ids.reshape(-1).astype(jnp.int32)
    out = _gather_rows(table, ids)
    return out.reshape(B, L, VOCAB)


# speedup vs baseline: 1.0117x; 1.0117x over previous
import functools

import jax
import jax.numpy as jnp
from jax import lax
from jax.experimental import pallas as pl
from jax.experimental.pallas import tpu as pltpu
from jax.experimental.pallas import tpu_sc as plsc

VOCAB = 1000
HIDDEN = 64
B = 1024
L = 50

NC = 2
NS = 16
NW = NC * NS

BATCHES_PER_W = B // NW


def _table_body(emb_ref, wt_ref, b_ref, out_ref):
    out_ref[...] = (
        jnp.dot(emb_ref[...], wt_ref[...], preferred_element_type=jnp.float32)
        + b_ref[...]
    )


def _build_table(emb_table, head_w_t, head_b_row):
    return pl.pallas_call(
        _table_body,
        out_shape=jax.ShapeDtypeStruct((VOCAB, VOCAB), jnp.float32),
    )(emb_table, head_w_t, head_b_row)


_mesh = plsc.VectorSubcoreMesh(
    core_axis_name="c", subcore_axis_name="s", num_cores=NC, num_subcores=NS
)


@functools.partial(
    pl.kernel,
    out_type=jax.ShapeDtypeStruct((B, L, VOCAB), jnp.float32),
    mesh=_mesh,
    scratch_types=[
        pltpu.VMEM((BATCHES_PER_W, L), jnp.int32),
        pltpu.VMEM((L, VOCAB), jnp.float32),
        pltpu.VMEM((L, VOCAB), jnp.float32),
        pltpu.SemaphoreType.DMA,
        pltpu.SemaphoreType.DMA,
        pltpu.SemaphoreType.DMA,
        pltpu.SemaphoreType.DMA,
    ],
    compiler_params=pltpu.CompilerParams(use_tc_tiling_on_sc=False),
)
def _gather_rows(table_hbm, idx_hbm, out_hbm, idx_v, buf0, buf1, gs0, gs1, ws0, ws1):
    wid = lax.axis_index("s") * NC + lax.axis_index("c")
    b_base = pl.multiple_of(wid * BATCHES_PER_W, BATCHES_PER_W)
    pltpu.sync_copy(idx_hbm.at[pl.ds(b_base, BATCHES_PER_W)], idx_v)

    bufs = (buf0, buf1)
    gsems = (gs0, gs1)
    wsems = (ws0, ws1)
    gather = [None, None]
    write = [None, None]

    gather[0] = pltpu.async_copy(table_hbm.at[idx_v.at[0]], bufs[0], gsems[0])
    for i in range(BATCHES_PER_W):
        p = i & 1
        gather[p].wait()
        write[p] = pltpu.async_copy(bufs[p], out_hbm.at[b_base + i], wsems[p])
        if i + 1 < BATCHES_PER_W:
            q = 1 - p
            if write[q] is not None:
                write[q].wait()
            gather[q] = pltpu.async_copy(
                table_hbm.at[idx_v.at[i + 1]], bufs[q], gsems[q]
            )
    write[0].wait()
    write[1].wait()


def kernel(input_ids, emb_table, head_w, head_b):
    table = _build_table(emb_table, head_w.T, head_b.reshape(1, VOCAB))
    ids = input_ids.astype(jnp.int32)
    return _gather_rows(table, ids)


# speedup vs baseline: 5.1602x; 5.1007x over previous
import functools

import jax
import jax.numpy as jnp
from jax import lax
from jax.experimental import pallas as pl
from jax.experimental.pallas import tpu as pltpu
from jax.experimental.pallas import tpu_sc as plsc

VOCAB = 1000
HIDDEN = 64
HPAD = 128
B = 1024
L = 50
N_ROWS = B * L

NC = 2
NS = 16
NW = NC * NS

ROWS_PER_W = N_ROWS // NW
CHUNK = 400
NCHUNK = ROWS_PER_W // CHUNK


_mesh = plsc.VectorSubcoreMesh(
    core_axis_name="c", subcore_axis_name="s", num_cores=NC, num_subcores=NS
)


@functools.partial(
    pl.kernel,
    out_type=jax.ShapeDtypeStruct((N_ROWS, HPAD), jnp.float32),
    mesh=_mesh,
    scratch_types=[
        pltpu.VMEM((ROWS_PER_W,), jnp.int32),
        pltpu.VMEM((CHUNK, HPAD), jnp.float32),
        pltpu.VMEM((CHUNK, HPAD), jnp.float32),
        pltpu.SemaphoreType.DMA,
        pltpu.SemaphoreType.DMA,
        pltpu.SemaphoreType.DMA,
        pltpu.SemaphoreType.DMA,
    ],
    compiler_params=pltpu.CompilerParams(use_tc_tiling_on_sc=False),
)
def _gather_embs(emb_hbm, idx_hbm, out_hbm, idx_v, buf0, buf1, gs0, gs1, ws0, ws1):
    wid = lax.axis_index("s") * NC + lax.axis_index("c")
    base = pl.multiple_of(wid * ROWS_PER_W, ROWS_PER_W)
    pltpu.sync_copy(idx_hbm.at[pl.ds(base, ROWS_PER_W)], idx_v)

    bufs = (buf0, buf1)
    gsems = (gs0, gs1)
    wsems = (ws0, ws1)
    gather = [None, None]
    write = [None, None]

    gather[0] = pltpu.async_copy(
        emb_hbm.at[idx_v.at[pl.ds(0, CHUNK)]], bufs[0], gsems[0]
    )
    for i in range(NCHUNK):
        p = i & 1
        gather[p].wait()
        write[p] = pltpu.async_copy(
            bufs[p], out_hbm.at[pl.ds(base + i * CHUNK, CHUNK)], wsems[p]
        )
        if i + 1 < NCHUNK:
            q = 1 - p
            if write[q] is not None:
                write[q].wait()
            gather[q] = pltpu.async_copy(
                emb_hbm.at[idx_v.at[pl.ds((i + 1) * CHUNK, CHUNK)]],
                bufs[q],
                gsems[q],
            )
    for w in write:
        if w is not None:
            w.wait()


def _head_body(embs_ref, w_ref, b_ref, out_ref):
    e = embs_ref[0][:, :HIDDEN]
    acc = lax.dot_general(
        w_ref[...], e, (((1,), (1,)), ((), ())),
        preferred_element_type=jnp.float32,
    )
    out_ref[0] = acc + b_ref[...]


def _head(embs3, head_w, head_b_col):
    return pl.pallas_call(
        _head_body,
        grid=(L,),
        in_specs=[
            pl.BlockSpec((1, B, HPAD), lambda l: (l, 0, 0)),
            pl.BlockSpec((VOCAB, HIDDEN), lambda l: (0, 0)),
            pl.BlockSpec((VOCAB, 1), lambda l: (0, 0)),
        ],
        out_specs=pl.BlockSpec((1, VOCAB, B), lambda l: (l, 0, 0)),
        out_shape=jax.ShapeDtypeStruct((L, VOCAB, B), jnp.float32),
    )(embs3, head_w, head_b_col)


def kernel(input_ids, emb_table, head_w, head_b):
    emb128 = jnp.pad(emb_table, ((0, 0), (0, HPAD - HIDDEN)))
    idx_flat = input_ids.astype(jnp.int32).T.reshape(-1)
    embs = _gather_embs(emb128, idx_flat)
    embs3 = embs.reshape(L, B, HPAD)
    planes = _head(embs3, head_w, head_b.reshape(VOCAB, 1))
    return jnp.transpose(planes, (2, 0, 1))


# speedup vs baseline: 5.1950x; 1.0067x over previous
import functools

import jax
import jax.numpy as jnp
from jax import lax
from jax.experimental import pallas as pl
from jax.experimental.pallas import tpu as pltpu
from jax.experimental.pallas import tpu_sc as plsc

VOCAB = 1000
HIDDEN = 64
HPAD = 128
B = 1024
L = 50
LHALF = L // 2

NC = 2
NS = 16
NW = NC * NS

HALF_ROWS = B * LHALF
ROWS_PER_W = HALF_ROWS // NW
CHUNK = 400
NCHUNK = ROWS_PER_W // CHUNK


_mesh = plsc.VectorSubcoreMesh(
    core_axis_name="c", subcore_axis_name="s", num_cores=NC, num_subcores=NS
)


@functools.partial(
    pl.kernel,
    out_type=jax.ShapeDtypeStruct((HALF_ROWS, HPAD), jnp.float32),
    mesh=_mesh,
    scratch_types=[
        pltpu.VMEM((ROWS_PER_W,), jnp.int32),
        pltpu.VMEM((CHUNK, HPAD), jnp.float32),
        pltpu.VMEM((CHUNK, HPAD), jnp.float32),
        pltpu.SemaphoreType.DMA,
        pltpu.SemaphoreType.DMA,
        pltpu.SemaphoreType.DMA,
        pltpu.SemaphoreType.DMA,
    ],
    compiler_params=pltpu.CompilerParams(use_tc_tiling_on_sc=False),
)
def _gather_embs(emb_hbm, idx_hbm, out_hbm, idx_v, buf0, buf1, gs0, gs1, ws0, ws1):
    wid = lax.axis_index("s") * NC + lax.axis_index("c")
    base = pl.multiple_of(wid * ROWS_PER_W, ROWS_PER_W)
    pltpu.sync_copy(idx_hbm.at[pl.ds(base, ROWS_PER_W)], idx_v)

    bufs = (buf0, buf1)
    gsems = (gs0, gs1)
    wsems = (ws0, ws1)
    gather = [None, None]
    write = [None, None]

    gather[0] = pltpu.async_copy(
        emb_hbm.at[idx_v.at[pl.ds(0, CHUNK)]], bufs[0], gsems[0]
    )
    for i in range(NCHUNK):
        p = i & 1
        gather[p].wait()
        write[p] = pltpu.async_copy(
            bufs[p], out_hbm.at[pl.ds(base + i * CHUNK, CHUNK)], wsems[p]
        )
        if i + 1 < NCHUNK:
            q = 1 - p
            if write[q] is not None:
                write[q].wait()
            gather[q] = pltpu.async_copy(
                emb_hbm.at[idx_v.at[pl.ds((i + 1) * CHUNK, CHUNK)]],
                bufs[q],
                gsems[q],
            )
    for w in write:
        if w is not None:
            w.wait()


def _head_body(embs_ref, w_ref, b_ref, out_ref):
    e = embs_ref[0][:, :HIDDEN]
    acc = lax.dot_general(
        w_ref[...], e, (((1,), (1,)), ((), ())),
        preferred_element_type=jnp.float32,
    )
    out_ref[0] = acc + b_ref[...]


def _head_lo(embs3, head_w, head_b_col):
    return pl.pallas_call(
        _head_body,
        grid=(LHALF,),
        in_specs=[
            pl.BlockSpec((1, B, HPAD), lambda l: (l, 0, 0)),
            pl.BlockSpec((VOCAB, HIDDEN), lambda l: (0, 0)),
            pl.BlockSpec((VOCAB, 1), lambda l: (0, 0)),
        ],
        out_specs=pl.BlockSpec((1, VOCAB, B), lambda l: (l, 0, 0)),
        out_shape=jax.ShapeDtypeStruct((L, VOCAB, B), jnp.float32),
    )(embs3, head_w, head_b_col)


def _head_hi_body(prev_ref, embs_ref, w_ref, b_ref, out_ref):
    del prev_ref
    _head_body(embs_ref, w_ref, b_ref, out_ref)


def _head_hi(prev, embs3, head_w, head_b_col):
    return pl.pallas_call(
        _head_hi_body,
        grid=(LHALF,),
        in_specs=[
            pl.BlockSpec(memory_space=pl.ANY),
            pl.BlockSpec((1, B, HPAD), lambda l: (l, 0, 0)),
            pl.BlockSpec((VOCAB, HIDDEN), lambda l: (0, 0)),
            pl.BlockSpec((VOCAB, 1), lambda l: (0, 0)),
        ],
        out_specs=pl.BlockSpec((1, VOCAB, B), lambda l: (l + LHALF, 0, 0)),
        out_shape=jax.ShapeDtypeStruct((L, VOCAB, B), jnp.float32),
        input_output_aliases={0: 0},
    )(prev, embs3, head_w, head_b_col)


def kernel(input_ids, emb_table, head_w, head_b):
    emb128 = jnp.pad(emb_table, ((0, 0), (0, HPAD - HIDDEN)))
    ids_t = input_ids.astype(jnp.int32).T
    idx_lo = ids_t[:LHALF].reshape(-1)
    idx_hi = ids_t[LHALF:].reshape(-1)
    embs_lo = _gather_embs(emb128, idx_lo).reshape(LHALF, B, HPAD)
    embs_hi = _gather_embs(emb128, idx_hi).reshape(LHALF, B, HPAD)
    head_b_col = head_b.reshape(VOCAB, 1)
    planes = _head_lo(embs_lo, head_w, head_b_col)
    planes = _head_hi(planes, embs_hi, head_w, head_b_col)
    return jnp.transpose(planes, (2, 0, 1))
